# baseline (speedup 1.0000x reference)
import functools

import jax
import jax.numpy as jnp
from jax import lax
from jax.experimental import pallas as pl
from jax.experimental.pallas import tpu as pltpu
from jax.experimental.pallas import tpu_sc as plsc

F = 26
V = 100000
E = 32
B = 4096
HID = 128

_info = plsc.get_sparse_core_info()
NC = _info.num_cores
NS = _info.num_subcores
NW = NC * NS
BPW = B // NW
RPW = F * BPW

EG = 16


TCH = 4096
NCH = V // TCH
TTAIL = V - NCH * TCH


def _tr_body(x_ref, o_ref, scratch):
    def chunk(c, carry):
        scratch[0:TCH, :] = jnp.transpose(x_ref[0, :, pl.ds(c * TCH, TCH)])
        o_ref[0, pl.ds(c * (TCH // 8), TCH // 8), :] = jnp.concatenate(
            [scratch[pl.Slice(m, TCH // 8, 8), :] for m in range(8)], axis=1
        )
        return carry

    lax.fori_loop(0, NCH, chunk, 0)
    scratch[0:TTAIL, :] = jnp.transpose(x_ref[0, :, pl.ds(NCH * TCH, TTAIL)])
    o_ref[0, pl.ds(NCH * (TCH // 8), TTAIL // 8), :] = jnp.concatenate(
        [scratch[pl.Slice(m, TTAIL // 8, 8), :] for m in range(8)], axis=1
    )


def _to_rows(tables_t):
    flat = pl.pallas_call(
        _tr_body,
        grid=(F, E // EG),
        in_specs=[pl.BlockSpec((1, EG, V), lambda f, g: (f, g, 0))],
        out_specs=pl.BlockSpec(
            (1, V // 8, 8 * EG), lambda f, g: (2 * f + g, 0, 0)
        ),
        out_shape=jax.ShapeDtypeStruct((F * 2, V // 8, 8 * EG), jnp.float32),
        scratch_shapes=[pltpu.VMEM((TCH, EG), jnp.float32)],
        compiler_params=pltpu.CompilerParams(
            vmem_limit_bytes=56 * 1024 * 1024
        ),
    )(tables_t)
    return flat.reshape(F * 2 * V, EG)


def _sc_gather(flat_tables, categorical_vars):
    mesh = plsc.VectorSubcoreMesh(core_axis_name="c", subcore_axis_name="s")

    @functools.partial(
        pl.kernel,
        mesh=mesh,
        out_type=jax.ShapeDtypeStruct((B, F * E), jnp.float32),
        scratch_types=[
            pltpu.VMEM((F, BPW), jnp.int32),
            pltpu.VMEM((F, BPW), jnp.int32),
            pltpu.VMEM((F, BPW), jnp.int32),
            pltpu.VMEM((RPW, EG), jnp.float32),
            pltpu.VMEM((RPW, EG), jnp.float32),
            pltpu.SemaphoreType.DMA,
            pltpu.SemaphoreType.DMA,
        ],
        compiler_params=pltpu.CompilerParams(use_tc_tiling_on_sc=False),
    )
    def k(tbl_hbm, idx_hbm, out_hbm, idx_raw, pidx0, pidx1, rows0, rows1,
          gsem, wsem):
        wid = lax.axis_index("s") * NC + lax.axis_index("c")
        b0 = wid * BPW
        pltpu.sync_copy(idx_hbm.at[:, pl.ds(b0, BPW)], idx_raw)

        n_chunk = BPW // 16

        def off_body(i, carry):
            f = i // n_chunk
            c = i - f * n_chunk
            sl = pl.ds(c * 16, 16)
            v = idx_raw[f, sl]
            pidx0[f, sl] = v + (2 * f) * V
            pidx1[f, sl] = v + (2 * f + 1) * V
            return carry

        lax.fori_loop(0, F * n_chunk, off_body, 0)

        def g_body(j, carry):
            sl = pl.ds(j * BPW, BPW)
            pltpu.make_async_copy(
                tbl_hbm.at[pidx0.at[j]], rows0.at[sl], gsem
            ).start()
            pltpu.make_async_copy(
                tbl_hbm.at[pidx1.at[j]], rows1.at[sl], gsem
            ).start()
            return carry

        lax.fori_loop(0, F, g_body, 0)

        def gw_body(j, carry):
            sl = pl.ds(j * BPW, BPW)
            pltpu.make_async_copy(
                tbl_hbm.at[pidx0.at[j]], rows0.at[sl], gsem
            ).wait()
            pltpu.make_async_copy(
                tbl_hbm.at[pidx1.at[j]], rows1.at[sl], gsem
            ).wait()
            return carry

        lax.fori_loop(0, F, gw_body, 0)

        def w_body(j, carry):
            sl = pl.ds(j * BPW, BPW)
            pltpu.make_async_copy(
                rows0.at[sl],
                out_hbm.at[pl.ds(b0, BPW), pl.ds(j * E, EG)],
                wsem,
            ).start()
            pltpu.make_async_copy(
                rows1.at[sl],
                out_hbm.at[pl.ds(b0, BPW), pl.ds(j * E + EG, EG)],
                wsem,
            ).start()
            return carry

        lax.fori_loop(0, F, w_body, 0)

        def ww_body(j, carry):
            sl = pl.ds(j * BPW, BPW)
            pltpu.make_async_copy(
                rows0.at[sl],
                out_hbm.at[pl.ds(b0, BPW), pl.ds(j * E, EG)],
                wsem,
            ).wait()
            pltpu.make_async_copy(
                rows1.at[sl],
                out_hbm.at[pl.ds(b0, BPW), pl.ds(j * E + EG, EG)],
                wsem,
            ).wait()
            return carry

        lax.fori_loop(0, F, ww_body, 0)

    return k(flat_tables, categorical_vars)


def _mlp_body(x_ref, w1_ref, b1_ref, w2_ref, b2_ref, o_ref):
    h = jnp.dot(x_ref[...], w1_ref[...], preferred_element_type=jnp.float32)
    h = jnp.maximum(h + b1_ref[...], 0.0)
    o = jnp.dot(h, w2_ref[...], preferred_element_type=jnp.float32)
    o_ref[...] = o + b2_ref[...]


def _mlp(cond, W1, b1, W2, b2):
    nblk = 8
    rows = B // nblk
    return pl.pallas_call(
        _mlp_body,
        grid=(nblk,),
        in_specs=[
            pl.BlockSpec((rows, F * E), lambda i: (i, 0)),
            pl.BlockSpec((F * E, HID), lambda i: (0, 0)),
            pl.BlockSpec((1, HID), lambda i: (0, 0)),
            pl.BlockSpec((HID, E), lambda i: (0, 0)),
            pl.BlockSpec((1, E), lambda i: (0, 0)),
        ],
        out_specs=pl.BlockSpec((rows, E), lambda i: (i, 0)),
        out_shape=jax.ShapeDtypeStruct((B, E), jnp.float32),
    )(cond, W1, b1.reshape(1, HID), W2, b2.reshape(1, E))


def kernel(categorical_vars, tables, W1, b1, W2, b2):
    tables_t = jnp.swapaxes(tables, 1, 2)
    tbl16 = _to_rows(tables_t)
    cond = _sc_gather(tbl16, categorical_vars)
    return _mlp(cond, W1, b1, W2, b2)


# speedup vs baseline: 2.3045x; 2.3045x over previous
import functools

import jax
import jax.numpy as jnp
from jax import lax
from jax.experimental import pallas as pl
from jax.experimental.pallas import tpu as pltpu
from jax.experimental.pallas import tpu_sc as plsc

F = 26
V = 100000
E = 32
B = 4096
HID = 128

_info = plsc.get_sparse_core_info()
NC = _info.num_cores
NS = _info.num_subcores
NW = NC * NS
BPW = B // NW
RPW = F * BPW

TCH = 4096
NCH = V // TCH
TTAIL = V - NCH * TCH


def _tr_body(x_ref, o_ref, scratch):
    eye = jnp.eye(E, dtype=jnp.float32)

    def emit(c, n):
        xs = x_ref[0, :, pl.ds(c * TCH, n)]
        scratch[pl.ds(0, n), :] = lax.dot_general(
            xs, eye, (((0,), (0,)), ((), ())),
            preferred_element_type=jnp.float32,
        )
        o_ref[0, pl.ds(c * (TCH // 4), n // 4), :] = jnp.concatenate(
            [scratch[pl.Slice(m, n // 4, 4), :] for m in range(4)], axis=1
        )

    def chunk(c, carry):
        emit(c, TCH)
        return carry

    lax.fori_loop(0, NCH, chunk, 0)
    emit(NCH, TTAIL)


def _to_rows(tables_t):
    return pl.pallas_call(
        _tr_body,
        grid=(F,),
        in_specs=[pl.BlockSpec((1, E, V), lambda f: (f, 0, 0))],
        out_specs=pl.BlockSpec((1, V // 4, 4 * E), lambda f: (f, 0, 0)),
        out_shape=jax.ShapeDtypeStruct((F, V // 4, 4 * E), jnp.float32),
        scratch_shapes=[pltpu.VMEM((TCH, E), jnp.float32)],
        compiler_params=pltpu.CompilerParams(
            vmem_limit_bytes=60 * 1024 * 1024
        ),
    )(tables_t)


def _sc_gather(flat_tables, categorical_vars):
    mesh = plsc.VectorSubcoreMesh(core_axis_name="c", subcore_axis_name="s")

    @functools.partial(
        pl.kernel,
        mesh=mesh,
        out_type=jax.ShapeDtypeStruct((B, F * E), jnp.float32),
        scratch_types=[
            pltpu.VMEM((F, BPW), jnp.int32),
            pltpu.VMEM((F, BPW), jnp.int32),
            pltpu.VMEM((RPW, E), jnp.float32),
            pltpu.SemaphoreType.DMA,
            pltpu.SemaphoreType.DMA,
        ],
        compiler_params=pltpu.CompilerParams(use_tc_tiling_on_sc=False),
    )
    def k(tbl_hbm, idx_hbm, out_hbm, idx_raw, pidx, rows, gsem, wsem):
        wid = lax.axis_index("s") * NC + lax.axis_index("c")
        b0 = wid * BPW
        pltpu.sync_copy(idx_hbm.at[:, pl.ds(b0, BPW)], idx_raw)

        n_chunk = BPW // 16

        def off_body(i, carry):
            f = i // n_chunk
            c = i - f * n_chunk
            sl = pl.ds(c * 16, 16)
            pidx[f, sl] = idx_raw[f, sl] + f * V
            return carry

        lax.fori_loop(0, F * n_chunk, off_body, 0)

        def g_body(j, carry):
            pltpu.make_async_copy(
                tbl_hbm.at[pidx.at[j]], rows.at[pl.ds(j * BPW, BPW)], gsem
            ).start()
            return carry

        lax.fori_loop(0, F, g_body, 0)

        def gw_body(j, carry):
            pltpu.make_async_copy(
                tbl_hbm.at[pidx.at[j]], rows.at[pl.ds(j * BPW, BPW)], gsem
            ).wait()
            return carry

        lax.fori_loop(0, F, gw_body, 0)

        def w_body(j, carry):
            pltpu.make_async_copy(
                rows.at[pl.ds(j * BPW, BPW)],
                out_hbm.at[pl.ds(b0, BPW), pl.ds(j * E, E)],
                wsem,
            ).start()
            return carry

        lax.fori_loop(0, F, w_body, 0)

        def ww_body(j, carry):
            pltpu.make_async_copy(
                rows.at[pl.ds(j * BPW, BPW)],
                out_hbm.at[pl.ds(b0, BPW), pl.ds(j * E, E)],
                wsem,
            ).wait()
            return carry

        lax.fori_loop(0, F, ww_body, 0)

    return k(flat_tables, categorical_vars)


def _mlp_body(x_ref, w1_ref, b1_ref, w2_ref, b2_ref, o_ref):
    h = jnp.dot(x_ref[...], w1_ref[...], preferred_element_type=jnp.float32)
    h = jnp.maximum(h + b1_ref[...], 0.0)
    o = jnp.dot(h, w2_ref[...], preferred_element_type=jnp.float32)
    o_ref[...] = o + b2_ref[...]


def _mlp(cond, W1, b1, W2, b2):
    nblk = 8
    rows = B // nblk
    return pl.pallas_call(
        _mlp_body,
        grid=(nblk,),
        in_specs=[
            pl.BlockSpec((rows, F * E), lambda i: (i, 0)),
            pl.BlockSpec((F * E, HID), lambda i: (0, 0)),
            pl.BlockSpec((1, HID), lambda i: (0, 0)),
            pl.BlockSpec((HID, E), lambda i: (0, 0)),
            pl.BlockSpec((1, E), lambda i: (0, 0)),
        ],
        out_specs=pl.BlockSpec((rows, E), lambda i: (i, 0)),
        out_shape=jax.ShapeDtypeStruct((B, E), jnp.float32),
    )(cond, W1, b1.reshape(1, HID), W2, b2.reshape(1, E))


def kernel(categorical_vars, tables, W1, b1, W2, b2):
    tables_t = jnp.swapaxes(tables, 1, 2)
    t3 = _to_rows(tables_t)
    flat = t3.reshape(F * (V // 4), 4 * E)
    tbl2d = flat.reshape(F * V, E)
    cond = _sc_gather(tbl2d, categorical_vars)
    return _mlp(cond, W1, b1, W2, b2)


# speedup vs baseline: 2.5640x; 1.1126x over previous
import functools

import jax
import jax.numpy as jnp
from jax import lax
from jax.experimental import pallas as pl
from jax.experimental.pallas import tpu as pltpu
from jax.experimental.pallas import tpu_sc as plsc

F = 26
V = 100000
E = 32
B = 4096
HID = 128

_info = plsc.get_sparse_core_info()
NC = _info.num_cores
NS = _info.num_subcores
NW = NC * NS
BPW = B // NW
RPW = F * BPW

TCH = 4096
NCH = V // TCH
TTAIL = V - NCH * TCH


def _tr_body(x_ref, o_ref, scratch):
    eye = jnp.eye(E, dtype=jnp.float32)

    def emit(c, n, buf):
        xs = x_ref[0, :, pl.ds(c * TCH, n)]
        scratch[buf, pl.ds(0, n), :] = lax.dot_general(
            xs, eye, (((0,), (0,)), ((), ())),
            preferred_element_type=jnp.float32,
        )
        o_ref[0, pl.ds(c * (TCH // 4), n // 4), :] = jnp.concatenate(
            [scratch[buf, pl.Slice(m, n // 4, 4), :] for m in range(4)],
            axis=1,
        )

    def chunk(i, carry):
        emit(2 * i, TCH, 0)
        emit(2 * i + 1, TCH, 1)
        return carry

    lax.fori_loop(0, NCH // 2, chunk, 0)
    emit(NCH, TTAIL, 0)


def _to_rows(tables_t):
    return pl.pallas_call(
        _tr_body,
        grid=(F,),
        in_specs=[pl.BlockSpec((1, E, V), lambda f: (f, 0, 0))],
        out_specs=pl.BlockSpec((1, V // 4, 4 * E), lambda f: (f, 0, 0)),
        out_shape=jax.ShapeDtypeStruct((F, V // 4, 4 * E), jnp.float32),
        scratch_shapes=[pltpu.VMEM((2, TCH, E), jnp.float32)],
        compiler_params=pltpu.CompilerParams(
            vmem_limit_bytes=60 * 1024 * 1024
        ),
    )(tables_t)


def _sc_gather(flat_tables, categorical_vars):
    mesh = plsc.VectorSubcoreMesh(core_axis_name="c", subcore_axis_name="s")

    @functools.partial(
        pl.kernel,
        mesh=mesh,
        out_type=jax.ShapeDtypeStruct((B, F * E), jnp.float32),
        scratch_types=[
            pltpu.VMEM((F, BPW), jnp.int32),
            pltpu.VMEM((F, BPW), jnp.int32),
            pltpu.VMEM((RPW, E), jnp.float32),
            pltpu.SemaphoreType.DMA,
            pltpu.SemaphoreType.DMA,
        ],
        compiler_params=pltpu.CompilerParams(use_tc_tiling_on_sc=False),
    )
    def k(tbl_hbm, idx_hbm, out_hbm, idx_raw, pidx, rows, gsem, wsem):
        wid = lax.axis_index("s") * NC + lax.axis_index("c")
        b0 = wid * BPW
        pltpu.sync_copy(idx_hbm.at[:, pl.ds(b0, BPW)], idx_raw)

        n_chunk = BPW // 16

        def off_body(i, carry):
            f = i // n_chunk
            c = i - f * n_chunk
            sl = pl.ds(c * 16, 16)
            pidx[f, sl] = idx_raw[f, sl] + f * V
            return carry

        lax.fori_loop(0, F * n_chunk, off_body, 0)

        def g_body(j, carry):
            pltpu.make_async_copy(
                tbl_hbm.at[pidx.at[j]], rows.at[pl.ds(j * BPW, BPW)], gsem
            ).start()
            return carry

        lax.fori_loop(0, F, g_body, 0)

        def gw_body(j, carry):
            pltpu.make_async_copy(
                tbl_hbm.at[pidx.at[j]], rows.at[pl.ds(j * BPW, BPW)], gsem
            ).wait()
            return carry

        lax.fori_loop(0, F, gw_body, 0)

        def w_body(j, carry):
            pltpu.make_async_copy(
                rows.at[pl.ds(j * BPW, BPW)],
                out_hbm.at[pl.ds(b0, BPW), pl.ds(j * E, E)],
                wsem,
            ).start()
            return carry

        lax.fori_loop(0, F, w_body, 0)

        def ww_body(j, carry):
            pltpu.make_async_copy(
                rows.at[pl.ds(j * BPW, BPW)],
                out_hbm.at[pl.ds(b0, BPW), pl.ds(j * E, E)],
                wsem,
            ).wait()
            return carry

        lax.fori_loop(0, F, ww_body, 0)

    return k(flat_tables, categorical_vars)


def _mlp_body(x_ref, w1_ref, b1_ref, w2_ref, b2_ref, o_ref):
    h = jnp.dot(x_ref[...], w1_ref[...], preferred_element_type=jnp.float32)
    h = jnp.maximum(h + b1_ref[...], 0.0)
    o = jnp.dot(h, w2_ref[...], preferred_element_type=jnp.float32)
    o_ref[...] = o + b2_ref[...]


def _mlp(cond, W1, b1, W2, b2):
    nblk = 8
    rows = B // nblk
    return pl.pallas_call(
        _mlp_body,
        grid=(nblk,),
        in_specs=[
            pl.BlockSpec((rows, F * E), lambda i: (i, 0)),
            pl.BlockSpec((F * E, HID), lambda i: (0, 0)),
            pl.BlockSpec((1, HID), lambda i: (0, 0)),
            pl.BlockSpec((HID, E), lambda i: (0, 0)),
            pl.BlockSpec((1, E), lambda i: (0, 0)),
        ],
        out_specs=pl.BlockSpec((rows, E), lambda i: (i, 0)),
        out_shape=jax.ShapeDtypeStruct((B, E), jnp.float32),
    )(cond, W1, b1.reshape(1, HID), W2, b2.reshape(1, E))


def kernel(categorical_vars, tables, W1, b1, W2, b2):
    tables_t = jnp.swapaxes(tables, 1, 2)
    t3 = _to_rows(tables_t)
    flat = t3.reshape(F * (V // 4), 4 * E)
    tbl2d = flat.reshape(F * V, E)
    cond = _sc_gather(tbl2d, categorical_vars)
    return _mlp(cond, W1, b1, W2, b2)


# speedup vs baseline: 2.6175x; 1.0209x over previous
import functools

import jax
import jax.numpy as jnp
from jax import lax
from jax.experimental import pallas as pl
from jax.experimental.pallas import tpu as pltpu
from jax.experimental.pallas import tpu_sc as plsc

F = 26
V = 100000
E = 32
B = 4096
HID = 128

_info = plsc.get_sparse_core_info()
NC = _info.num_cores
NS = _info.num_subcores
NW = NC * NS
BPW = B // NW
RPW = F * BPW

TCH = 4096
NCH = V // TCH
TTAIL = V - NCH * TCH


def _tr_body(x_ref, o_ref, scratch):
    eye = jnp.eye(E, dtype=jnp.float32)

    def emit(c, n, buf):
        xs = x_ref[0, :, pl.ds(c * TCH, n)]
        scratch[buf, pl.ds(0, n), :] = lax.dot_general(
            xs, eye, (((0,), (0,)), ((), ())),
            preferred_element_type=jnp.float32,
        )
        o_ref[0, pl.ds(c * (TCH // 4), n // 4), :] = jnp.concatenate(
            [scratch[buf, pl.Slice(m, n // 4, 4), :] for m in range(4)],
            axis=1,
        )

    def chunk(i, carry):
        for u in range(4):
            emit(4 * i + u, TCH, u)
        return carry

    lax.fori_loop(0, NCH // 4, chunk, 0)
    emit(NCH, TTAIL, 0)


def _to_rows(tables_t):
    return pl.pallas_call(
        _tr_body,
        grid=(F,),
        in_specs=[pl.BlockSpec((1, E, V), lambda f: (f, 0, 0))],
        out_specs=pl.BlockSpec((1, V // 4, 4 * E), lambda f: (f, 0, 0)),
        out_shape=jax.ShapeDtypeStruct((F, V // 4, 4 * E), jnp.float32),
        scratch_shapes=[pltpu.VMEM((4, TCH, E), jnp.float32)],
        compiler_params=pltpu.CompilerParams(
            vmem_limit_bytes=62 * 1024 * 1024
        ),
    )(tables_t)


def _sc_gather(flat_tables, categorical_vars):
    mesh = plsc.VectorSubcoreMesh(core_axis_name="c", subcore_axis_name="s")

    @functools.partial(
        pl.kernel,
        mesh=mesh,
        out_type=jax.ShapeDtypeStruct((B, F * E), jnp.float32),
        scratch_types=[
            pltpu.VMEM((F, BPW), jnp.int32),
            pltpu.VMEM((F, BPW), jnp.int32),
            pltpu.VMEM((RPW, E), jnp.float32),
            pltpu.SemaphoreType.DMA,
            pltpu.SemaphoreType.DMA,
        ],
        compiler_params=pltpu.CompilerParams(use_tc_tiling_on_sc=False),
    )
    def k(tbl_hbm, idx_hbm, out_hbm, idx_raw, pidx, rows, gsem, wsem):
        wid = lax.axis_index("s") * NC + lax.axis_index("c")
        b0 = wid * BPW
        pltpu.sync_copy(idx_hbm.at[:, pl.ds(b0, BPW)], idx_raw)

        n_chunk = BPW // 16

        def off_body(i, carry):
            f = i // n_chunk
            c = i - f * n_chunk
            sl = pl.ds(c * 16, 16)
            pidx[f, sl] = idx_raw[f, sl] + f * V
            return carry

        lax.fori_loop(0, F * n_chunk, off_body, 0)

        def g_body(j, carry):
            pltpu.make_async_copy(
                tbl_hbm.at[pidx.at[j]], rows.at[pl.ds(j * BPW, BPW)], gsem
            ).start()
            return carry

        lax.fori_loop(0, F, g_body, 0)

        def gw_body(j, carry):
            pltpu.make_async_copy(
                tbl_hbm.at[pidx.at[j]], rows.at[pl.ds(j * BPW, BPW)], gsem
            ).wait()
            return carry

        lax.fori_loop(0, F, gw_body, 0)

        def w_body(j, carry):
            pltpu.make_async_copy(
                rows.at[pl.ds(j * BPW, BPW)],
                out_hbm.at[pl.ds(b0, BPW), pl.ds(j * E, E)],
                wsem,
            ).start()
            return carry

        lax.fori_loop(0, F, w_body, 0)

        def ww_body(j, carry):
            pltpu.make_async_copy(
                rows.at[pl.ds(j * BPW, BPW)],
                out_hbm.at[pl.ds(b0, BPW), pl.ds(j * E, E)],
                wsem,
            ).wait()
            return carry

        lax.fori_loop(0, F, ww_body, 0)

    return k(flat_tables, categorical_vars)


def _mlp_body(x_ref, w1_ref, b1_ref, w2_ref, b2_ref, o_ref):
    h = jnp.dot(x_ref[...], w1_ref[...], preferred_element_type=jnp.float32)
    h = jnp.maximum(h + b1_ref[...], 0.0)
    o = jnp.dot(h, w2_ref[...], preferred_element_type=jnp.float32)
    o_ref[...] = o + b2_ref[...]


def _mlp(cond, W1, b1, W2, b2):
    nblk = 8
    rows = B // nblk
    return pl.pallas_call(
        _mlp_body,
        grid=(nblk,),
        in_specs=[
            pl.BlockSpec((rows, F * E), lambda i: (i, 0)),
            pl.BlockSpec((F * E, HID), lambda i: (0, 0)),
            pl.BlockSpec((1, HID), lambda i: (0, 0)),
            pl.BlockSpec((HID, E), lambda i: (0, 0)),
            pl.BlockSpec((1, E), lambda i: (0, 0)),
        ],
        out_specs=pl.BlockSpec((rows, E), lambda i: (i, 0)),
        out_shape=jax.ShapeDtypeStruct((B, E), jnp.float32),
    )(cond, W1, b1.reshape(1, HID), W2, b2.reshape(1, E))


def kernel(categorical_vars, tables, W1, b1, W2, b2):
    tables_t = jnp.swapaxes(tables, 1, 2)
    t3 = _to_rows(tables_t)
    flat = t3.reshape(F * (V // 4), 4 * E)
    tbl2d = flat.reshape(F * V, E)
    cond = _sc_gather(tbl2d, categorical_vars)
    return _mlp(cond, W1, b1, W2, b2)
